# baseline (speedup 1.0000x reference)
import functools

import jax
import jax.numpy as jnp
from jax import lax
from jax.experimental import pallas as pl
from jax.experimental.pallas import tpu as pltpu
from jax.experimental.pallas import tpu_sc as plsc

N_RAYS = 32768
N_SURF = 64
NC = 2
NS = 16
NW = NC * NS
L = 16
R = N_RAYS // NW
G = R // L

_MESH = plsc.VectorSubcoreMesh(core_axis_name="c", subcore_axis_name="s")
_PARAMS = pltpu.CompilerParams(needs_layout_passes=False,
                               use_tc_tiling_on_sc=False)


def _router_body(t_ref, bt_ref, si_ref, t_v, bt_v, si_v, sem0, sem1):
    wid = lax.axis_index("s") * NC + lax.axis_index("c")
    base = wid * R
    cp0 = pltpu.async_copy(t_ref.at[:, pl.ds(wid * 8, 4)], t_v.at[0], sem0)
    cp1 = pltpu.async_copy(t_ref.at[:, pl.ds(wid * 8 + 4, 4)], t_v.at[1], sem1)
    inf = jnp.float32(jnp.inf)

    def make_group(h):
        def group(g, carry):
            gg = h * (G // 2) + g
            cb = g // 8
            off = (g % 8) * L
            bts = [jnp.full((L,), inf, dtype=jnp.float32) for _ in range(8)]
            bps = [jnp.zeros((L,), dtype=jnp.int32) for _ in range(8)]
            for p in range(N_SURF // 8):
                pv = jnp.full((L,), p, jnp.int32)
                for k in range(8):
                    tv = t_v[h, p, cb, k, pl.ds(off, L)]
                    c = tv < bts[k]
                    bts[k] = jnp.where(c, tv, bts[k])
                    bps[k] = jnp.where(c, pv, bps[k])
            sis = [bps[k] * 8 + k for k in range(8)]

            def combine(ta, ia, tb, ib):
                c = (ta < tb) | ((ta == tb) & (ia < ib))
                return jnp.where(c, ta, tb), jnp.where(c, ia, ib)

            while len(bts) > 1:
                nt, ni = [], []
                for a in range(0, len(bts), 2):
                    tt, ii = combine(bts[a], sis[a], bts[a + 1], sis[a + 1])
                    nt.append(tt)
                    ni.append(ii)
                bts, sis = nt, ni
            sl = pl.ds(gg * L, L)
            bt_v[sl] = bts[0]
            si_v[sl] = sis[0]
            return carry
        return group

    cp0.wait()
    lax.fori_loop(0, G // 2, make_group(0), 0)
    cp1.wait()
    lax.fori_loop(0, G // 2, make_group(1), 0)

    pltpu.sync_copy(bt_v, bt_ref.at[pl.ds(base, R)])
    pltpu.sync_copy(si_v, si_ref.at[pl.ds(base, R)])


_router_kernel = functools.partial(
    pl.kernel,
    out_type=(jax.ShapeDtypeStruct((N_RAYS,), jnp.float32),
              jax.ShapeDtypeStruct((N_RAYS,), jnp.int32)),
    scratch_types=[
        pltpu.VMEM((2, 8, 4, 8, 128), jnp.float32),
        pltpu.VMEM((R,), jnp.float32),
        pltpu.VMEM((R,), jnp.int32),
        pltpu.SemaphoreType.DMA,
        pltpu.SemaphoreType.DMA,
    ],
    mesh=_MESH,
    compiler_params=_PARAMS,
)(_router_body)


def _epilogue_body(bt_ref, si_ref, px_ref, py_ref, pz_ref,
                   dx_ref, dy_ref, dz_ref, int_ref,
                   w0_ref, w1_ref, w2_ref, w3_ref, w4_ref, w5_ref,
                   w6_ref, w7_ref, w8_ref, dec_ref,
                   opx_ref, opy_ref, opz_ref, odx_ref, ody_ref, odz_ref,
                   oint_ref,
                   bt_v, si_v, p_v, d_v, int_v, w_v, dec_v, o_v, oint_v):
    w_refs = (w0_ref, w1_ref, w2_ref, w3_ref, w4_ref, w5_ref,
              w6_ref, w7_ref, w8_ref)
    wid = lax.axis_index("s") * NC + lax.axis_index("c")
    base = wid * R
    pltpu.sync_copy(bt_ref.at[pl.ds(base, R)], bt_v)
    pltpu.sync_copy(si_ref.at[pl.ds(base, R)], si_v)
    for c, ref in enumerate((px_ref, py_ref, pz_ref)):
        pltpu.sync_copy(ref.at[pl.ds(base, R)], p_v.at[c])
    for c, ref in enumerate((dx_ref, dy_ref, dz_ref)):
        pltpu.sync_copy(ref.at[pl.ds(base, R)], d_v.at[c])
    pltpu.sync_copy(int_ref.at[pl.ds(base, R)], int_v)
    for k, ref in enumerate(w_refs):
        pltpu.sync_copy(ref, w_v.at[k])
    pltpu.sync_copy(dec_ref, dec_v)
    inf = jnp.float32(jnp.inf)

    def group(g, carry):
        sl = pl.ds(g * L, L)
        bt = bt_v[sl]
        bi = si_v[sl]
        wg = [plsc.load_gather(w_v, [jnp.full((L,), k, jnp.int32), bi])
              for k in range(9)]
        dg = plsc.load_gather(dec_v, [bi])
        px = [p_v[c, sl] for c in range(3)]
        dx = [d_v[c, sl] for c in range(3)]
        it = int_v[sl]
        hit = (bt < inf) & (it > jnp.float32(0.0))
        for c in range(3):
            o_v[c, sl] = jnp.where(hit, px[c] + bt * dx[c], px[c])
            o_v[3 + c, sl] = jnp.where(
                hit, dx[0] * wg[c] + dx[1] * wg[3 + c] + dx[2] * wg[6 + c],
                dx[c])
        oint_v[sl] = jnp.where(hit, it * dg, it)
        return carry

    lax.fori_loop(0, G, group, 0)

    for c, ref in enumerate((opx_ref, opy_ref, opz_ref, odx_ref, ody_ref, odz_ref)):
        pltpu.sync_copy(o_v.at[c], ref.at[pl.ds(base, R)])
    pltpu.sync_copy(oint_v, oint_ref.at[pl.ds(base, R)])


_epilogue_kernel = functools.partial(
    pl.kernel,
    out_type=tuple([jax.ShapeDtypeStruct((N_RAYS,), jnp.float32)] * 7),
    scratch_types=[
        pltpu.VMEM((R,), jnp.float32),
        pltpu.VMEM((R,), jnp.int32),
        pltpu.VMEM((3, R), jnp.float32),
        pltpu.VMEM((3, R), jnp.float32),
        pltpu.VMEM((R,), jnp.float32),
        pltpu.VMEM((9, N_SURF), jnp.float32),
        pltpu.VMEM((N_SURF,), jnp.float32),
        pltpu.VMEM((6, R), jnp.float32),
        pltpu.VMEM((R,), jnp.float32),
    ],
    mesh=_MESH,
    compiler_params=_PARAMS,
)(_epilogue_body)


def kernel(pos, dir, intensity, t_matrix, W, decay, map_to_element, map_to_surface):
    del map_to_element, map_to_surface
    t4 = t_matrix.T.reshape(8, 8, 256, 128).transpose(0, 2, 1, 3)
    bt, si = _router_kernel(t4)
    opx, opy, opz, odx, ody, odz, oint = _epilogue_kernel(
        bt, si, pos[:, 0], pos[:, 1], pos[:, 2],
        dir[:, 0], dir[:, 1], dir[:, 2], intensity,
        W[:, 0, 0], W[:, 0, 1], W[:, 0, 2],
        W[:, 1, 0], W[:, 1, 1], W[:, 1, 2],
        W[:, 2, 0], W[:, 2, 1], W[:, 2, 2], decay)
    return (jnp.stack([opx, opy, opz], axis=1),
            jnp.stack([odx, ody, odz], axis=1), oint)


# speedup vs baseline: 1.0384x; 1.0384x over previous
import functools

import jax
import jax.numpy as jnp
from jax import lax
from jax.experimental import pallas as pl
from jax.experimental.pallas import tpu as pltpu
from jax.experimental.pallas import tpu_sc as plsc

N_RAYS = 32768
N_SURF = 64
NC = 2
NS = 16
NW = NC * NS
L = 16
R = N_RAYS // NW
G = R // L


def _scene_body(t_ref, px_ref, py_ref, pz_ref, dx_ref, dy_ref, dz_ref,
                int_ref, w0_ref, w1_ref, w2_ref, w3_ref, w4_ref, w5_ref,
                w6_ref, w7_ref, w8_ref, dec_ref,
                opx_ref, opy_ref, opz_ref, odx_ref, ody_ref, odz_ref, oint_ref,
                t_v, p_v, d_v, int_v, w_v, dec_v, o_v, oint_v, sem0, sem1):
    w_refs = (w0_ref, w1_ref, w2_ref, w3_ref, w4_ref, w5_ref,
              w6_ref, w7_ref, w8_ref)
    wid = lax.axis_index("s") * NC + lax.axis_index("c")
    base = wid * R
    cp0 = pltpu.async_copy(t_ref.at[:, pl.ds(wid * 8, 4)], t_v.at[0], sem0)
    cp1 = pltpu.async_copy(t_ref.at[:, pl.ds(wid * 8 + 4, 4)], t_v.at[1], sem1)
    for c, ref in enumerate((px_ref, py_ref, pz_ref)):
        pltpu.sync_copy(ref.at[pl.ds(base, R)], p_v.at[c])
    for c, ref in enumerate((dx_ref, dy_ref, dz_ref)):
        pltpu.sync_copy(ref.at[pl.ds(base, R)], d_v.at[c])
    pltpu.sync_copy(int_ref.at[pl.ds(base, R)], int_v)
    for k, ref in enumerate(w_refs):
        pltpu.sync_copy(ref, w_v.at[k])
    pltpu.sync_copy(dec_ref, dec_v)

    inf = jnp.float32(jnp.inf)

    def make_group(h):
        def group(g):
            gg = h * (G // 2) + g
            cb = g // 8
            off = (g % 8) * L
            bts = [jnp.full((L,), inf, dtype=jnp.float32) for _ in range(8)]
            bps = [jnp.zeros((L,), dtype=jnp.int32) for _ in range(8)]
            for p in range(N_SURF // 8):
                pv = jnp.full((L,), p, jnp.int32)
                for k in range(8):
                    tv = t_v[h, p, cb, k, pl.ds(off, L)]
                    c = tv < bts[k]
                    bts[k] = jnp.where(c, tv, bts[k])
                    bps[k] = jnp.where(c, pv, bps[k])
            sis = [bps[k] * 8 + k for k in range(8)]

            def combine(ta, ia, tb, ib):
                c = (ta < tb) | ((ta == tb) & (ia < ib))
                return jnp.where(c, ta, tb), jnp.where(c, ia, ib)

            while len(bts) > 1:
                nt, ni = [], []
                for a in range(0, len(bts), 2):
                    tt, ii = combine(bts[a], sis[a], bts[a + 1], sis[a + 1])
                    nt.append(tt)
                    ni.append(ii)
                bts, sis = nt, ni
            bt, bi = bts[0], sis[0]
            wg = [plsc.load_gather(w_v, [jnp.full((L,), k, jnp.int32), bi])
                  for k in range(9)]
            dg = plsc.load_gather(dec_v, [bi])
            sl = pl.ds(gg * L, L)
            px = [p_v[c, sl] for c in range(3)]
            dx = [d_v[c, sl] for c in range(3)]
            it = int_v[sl]
            hit = (bt < inf) & (it > jnp.float32(0.0))
            for c in range(3):
                o_v[c, sl] = jnp.where(hit, px[c] + bt * dx[c], px[c])
                o_v[3 + c, sl] = jnp.where(
                    hit, dx[0] * wg[c] + dx[1] * wg[3 + c] + dx[2] * wg[6 + c],
                    dx[c])
            oint_v[sl] = jnp.where(hit, it * dg, it)
        return group

    cp0.wait()
    plsc.parallel_loop(0, G // 2, unroll=2)(make_group(0))
    cp1.wait()
    plsc.parallel_loop(0, G // 2, unroll=2)(make_group(1))

    for c, ref in enumerate((opx_ref, opy_ref, opz_ref, odx_ref, ody_ref, odz_ref)):
        pltpu.sync_copy(o_v.at[c], ref.at[pl.ds(base, R)])
    pltpu.sync_copy(oint_v, oint_ref.at[pl.ds(base, R)])


_scene_kernel = functools.partial(
    pl.kernel,
    out_type=tuple([jax.ShapeDtypeStruct((N_RAYS,), jnp.float32)] * 7),
    scratch_types=[
        pltpu.VMEM((2, 8, 4, 8, 128), jnp.float32),
        pltpu.VMEM((3, R), jnp.float32),
        pltpu.VMEM((3, R), jnp.float32),
        pltpu.VMEM((R,), jnp.float32),
        pltpu.VMEM((9, N_SURF), jnp.float32),
        pltpu.VMEM((N_SURF,), jnp.float32),
        pltpu.VMEM((6, R), jnp.float32),
        pltpu.VMEM((R,), jnp.float32),
        pltpu.SemaphoreType.DMA,
        pltpu.SemaphoreType.DMA,
    ],
    mesh=plsc.VectorSubcoreMesh(core_axis_name="c", subcore_axis_name="s"),
    compiler_params=pltpu.CompilerParams(needs_layout_passes=False,
                                         use_tc_tiling_on_sc=False),
)(_scene_body)


def kernel(pos, dir, intensity, t_matrix, W, decay, map_to_element, map_to_surface):
    del map_to_element, map_to_surface
    t4 = t_matrix.T.reshape(8, 8, 256, 128).transpose(0, 2, 1, 3)
    opx, opy, opz, odx, ody, odz, oint = _scene_kernel(
        t4, pos[:, 0], pos[:, 1], pos[:, 2],
        dir[:, 0], dir[:, 1], dir[:, 2], intensity,
        W[:, 0, 0], W[:, 0, 1], W[:, 0, 2],
        W[:, 1, 0], W[:, 1, 1], W[:, 1, 2],
        W[:, 2, 0], W[:, 2, 1], W[:, 2, 2], decay)
    return (jnp.stack([opx, opy, opz], axis=1),
            jnp.stack([odx, ody, odz], axis=1), oint)


# speedup vs baseline: 1.0882x; 1.0480x over previous
import functools

import jax
import jax.numpy as jnp
from jax import lax
from jax.experimental import pallas as pl
from jax.experimental.pallas import tpu as pltpu
from jax.experimental.pallas import tpu_sc as plsc

N_RAYS = 32768
N_SURF = 64
NC = 2
NS = 16
NW = NC * NS
L = 16
R = N_RAYS // NW
G = R // L


def _scene_body(t_ref, px_ref, py_ref, pz_ref, dx_ref, dy_ref, dz_ref,
                int_ref, w0_ref, w1_ref, w2_ref, w3_ref, w4_ref, w5_ref,
                w6_ref, w7_ref, w8_ref, dec_ref,
                opx_ref, opy_ref, opz_ref, odx_ref, ody_ref, odz_ref, oint_ref,
                t_v, p_v, d_v, int_v, w_v, dec_v, o_v, oint_v, sem0, sem1):
    w_refs = (w0_ref, w1_ref, w2_ref, w3_ref, w4_ref, w5_ref,
              w6_ref, w7_ref, w8_ref)
    wid = lax.axis_index("s") * NC + lax.axis_index("c")
    base = wid * R
    cp0 = pltpu.async_copy(t_ref.at[:, pl.ds(wid * 8, 4)], t_v.at[0], sem0)
    cp1 = pltpu.async_copy(t_ref.at[:, pl.ds(wid * 8 + 4, 4)], t_v.at[1], sem1)
    for c, ref in enumerate((px_ref, py_ref, pz_ref)):
        pltpu.sync_copy(ref.at[pl.ds(base, R)], p_v.at[c])
    for c, ref in enumerate((dx_ref, dy_ref, dz_ref)):
        pltpu.sync_copy(ref.at[pl.ds(base, R)], d_v.at[c])
    pltpu.sync_copy(int_ref.at[pl.ds(base, R)], int_v)
    for k, ref in enumerate(w_refs):
        pltpu.sync_copy(ref, w_v.at[k])
    pltpu.sync_copy(dec_ref, dec_v)

    inf = jnp.float32(jnp.inf)

    def make_group(h):
        def group(g):
            gg = h * (G // 2) + g
            cb = g // 8
            off = (g % 8) * L
            bts = [jnp.full((L,), inf, dtype=jnp.float32) for _ in range(8)]
            bps = [jnp.zeros((L,), dtype=jnp.int32) for _ in range(8)]
            for p in range(N_SURF // 8):
                pv = jnp.full((L,), p, jnp.int32)
                for k in range(8):
                    tv = t_v[h, p, cb, k, pl.ds(off, L)]
                    c = tv < bts[k]
                    bts[k] = jnp.where(c, tv, bts[k])
                    bps[k] = jnp.where(c, pv, bps[k])
            sis = [bps[k] * 8 + k for k in range(8)]

            def combine(ta, ia, tb, ib):
                c = (ta < tb) | ((ta == tb) & (ia < ib))
                return jnp.where(c, ta, tb), jnp.where(c, ia, ib)

            while len(bts) > 1:
                nt, ni = [], []
                for a in range(0, len(bts), 2):
                    tt, ii = combine(bts[a], sis[a], bts[a + 1], sis[a + 1])
                    nt.append(tt)
                    ni.append(ii)
                bts, sis = nt, ni
            bt, bi = bts[0], sis[0]
            wg = [plsc.load_gather(w_v, [jnp.full((L,), k, jnp.int32), bi])
                  for k in range(9)]
            dg = plsc.load_gather(dec_v, [bi])
            sl = pl.ds(gg * L, L)
            px = [p_v[c, sl] for c in range(3)]
            dx = [d_v[c, sl] for c in range(3)]
            it = int_v[sl]
            hit = (bt < inf) & (it > jnp.float32(0.0))
            for c in range(3):
                o_v[c, sl] = jnp.where(hit, px[c] + bt * dx[c], px[c])
                o_v[3 + c, sl] = jnp.where(
                    hit, dx[0] * wg[c] + dx[1] * wg[3 + c] + dx[2] * wg[6 + c],
                    dx[c])
            oint_v[sl] = jnp.where(hit, it * dg, it)
        return group

    cp0.wait()
    plsc.parallel_loop(0, G // 2, unroll=1)(make_group(0))
    cp1.wait()
    plsc.parallel_loop(0, G // 2, unroll=1)(make_group(1))

    for c, ref in enumerate((opx_ref, opy_ref, opz_ref, odx_ref, ody_ref, odz_ref)):
        pltpu.sync_copy(o_v.at[c], ref.at[pl.ds(base, R)])
    pltpu.sync_copy(oint_v, oint_ref.at[pl.ds(base, R)])


_scene_kernel = functools.partial(
    pl.kernel,
    out_type=tuple([jax.ShapeDtypeStruct((N_RAYS,), jnp.float32)] * 7),
    scratch_types=[
        pltpu.VMEM((2, 8, 4, 8, 128), jnp.float32),
        pltpu.VMEM((3, R), jnp.float32),
        pltpu.VMEM((3, R), jnp.float32),
        pltpu.VMEM((R,), jnp.float32),
        pltpu.VMEM((9, N_SURF), jnp.float32),
        pltpu.VMEM((N_SURF,), jnp.float32),
        pltpu.VMEM((6, R), jnp.float32),
        pltpu.VMEM((R,), jnp.float32),
        pltpu.SemaphoreType.DMA,
        pltpu.SemaphoreType.DMA,
    ],
    mesh=plsc.VectorSubcoreMesh(core_axis_name="c", subcore_axis_name="s"),
    compiler_params=pltpu.CompilerParams(needs_layout_passes=False,
                                         use_tc_tiling_on_sc=False),
)(_scene_body)


def kernel(pos, dir, intensity, t_matrix, W, decay, map_to_element, map_to_surface):
    del map_to_element, map_to_surface
    t4 = t_matrix.T.reshape(8, 8, 256, 128).transpose(0, 2, 1, 3)
    opx, opy, opz, odx, ody, odz, oint = _scene_kernel(
        t4, pos[:, 0], pos[:, 1], pos[:, 2],
        dir[:, 0], dir[:, 1], dir[:, 2], intensity,
        W[:, 0, 0], W[:, 0, 1], W[:, 0, 2],
        W[:, 1, 0], W[:, 1, 1], W[:, 1, 2],
        W[:, 2, 0], W[:, 2, 1], W[:, 2, 2], decay)
    return (jnp.stack([opx, opy, opz], axis=1),
            jnp.stack([odx, ody, odz], axis=1), oint)
